# baseline (speedup 1.0000x reference)
import jax
import jax.numpy as jnp
from jax.experimental import pallas as pl
from jax.experimental.pallas import tpu as pltpu

_MEM = 100000
_D = 256
_NC = 10
_B = 1000
_NB = _MEM // (_NC * _B)
_CHUNK = _MEM // _NC


def _score(keys, q, qn2):
    dots = jnp.sum(keys * q, axis=1)
    kn2 = jnp.sum(keys * keys, axis=1)
    return dots * jnp.abs(dots) / jnp.maximum(qn2 * kn2, 1e-16)


def _chunk_update(i, c, q, qn2, ptr, keys, best_v, best_b):
    base = c * _CHUNK + i * _B
    s = _score(keys, q, qn2)

    def _upd(m):
        better = (m > best_v[0]) | ((m == best_v[0]) & (base < best_b[0]))

        @pl.when(better)
        def _():
            best_v[0] = m
            best_b[0] = base

    @pl.when(base + _B <= ptr)
    def _():
        _upd(jnp.max(s))

    @pl.when((base < ptr) & (ptr < base + _B))
    def _():
        gidx = base + jax.lax.iota(jnp.int32, _B)
        _upd(jnp.max(jnp.where(gidx < ptr, s, -jnp.inf)))


def _scan_body(ptr_ref, q_ref, *rest):
    mrefs = rest[:_NC]
    blk_ref, best_v, best_b = rest[_NC], rest[_NC + 1], rest[_NC + 2]
    i = pl.program_id(0)

    @pl.when(i == 0)
    def _():
        best_v[0] = -jnp.inf
        best_b[0] = 0

    q = q_ref[...]
    qn2 = jnp.sum(q * q)
    ptr = ptr_ref[0]
    for c, mref in enumerate(mrefs):
        _chunk_update(i, c, q, qn2, ptr, mref[...], best_v, best_b)

    @pl.when(i == pl.num_programs(0) - 1)
    def _():
        blk_ref[0] = best_b[0]


def _pick_body(s_ref, q_ref, keys_ref, mem_ref, out_ref, vsem):
    base = s_ref[0]
    ptr = s_ref[1]
    q = q_ref[...]
    qn2 = jnp.sum(q * q)
    s = _score(keys_ref[...], q, qn2)
    gidx = base + jax.lax.iota(jnp.int32, _B)
    s = jnp.where(gidx < ptr, s, -jnp.inf)
    li = jnp.argmax(s).astype(jnp.int32)
    gi = base + li
    copy = pltpu.make_async_copy(
        mem_ref.at[pl.ds(gi, 1), pl.ds(_D, _D)], out_ref, vsem)
    copy.start()
    copy.wait()


def kernel(query, memory, pointer):
    q2 = query.reshape(1, _D)
    ptr = jnp.asarray(pointer, jnp.int32).reshape(1)

    def _mspec(c):
        nblk = _CHUNK // _B
        return pl.BlockSpec((_B, _D), lambda i, p, c=c: (c * nblk + i, 0))

    blk = pl.pallas_call(
        _scan_body,
        grid_spec=pltpu.PrefetchScalarGridSpec(
            num_scalar_prefetch=1,
            grid=(_NB,),
            in_specs=[pl.BlockSpec((1, _D), lambda i, p: (0, 0))]
            + [_mspec(c) for c in range(_NC)],
            out_specs=pl.BlockSpec(memory_space=pltpu.SMEM),
            scratch_shapes=[
                pltpu.SMEM((1,), jnp.float32),
                pltpu.SMEM((1,), jnp.int32),
            ],
        ),
        out_shape=jax.ShapeDtypeStruct((1,), jnp.int32),
    )(ptr, q2, *([memory] * _NC))

    sarg = jnp.concatenate([blk, ptr])
    row = pl.pallas_call(
        _pick_body,
        grid_spec=pltpu.PrefetchScalarGridSpec(
            num_scalar_prefetch=1,
            grid=(1,),
            in_specs=[
                pl.BlockSpec((1, _D), lambda i, s: (0, 0)),
                pl.BlockSpec((_B, _D), lambda i, s: (s[0] // _B, 0)),
                pl.BlockSpec(memory_space=pl.ANY),
            ],
            out_specs=pl.BlockSpec((1, _D), lambda i, s: (0, 0)),
            scratch_shapes=[pltpu.SemaphoreType.DMA],
        ),
        out_shape=jax.ShapeDtypeStruct((1, _D), jnp.float32),
    )(sarg, q2, memory, memory)

    return row.reshape(_D)


# speedup vs baseline: 1.2778x; 1.2778x over previous
import jax
import jax.numpy as jnp
from jax.experimental import pallas as pl
from jax.experimental.pallas import tpu as pltpu

_MEM = 100000
_D = 256
_NC = 10
_B = 1000
_NB = _MEM // (_NC * _B)
_CHUNK = _MEM // _NC


def _score(keys, q, qn2):
    dots = jnp.sum(keys * q, axis=1)
    kn2 = jnp.sum(keys * keys, axis=1)
    return dots * jnp.abs(dots) / jnp.maximum(qn2 * kn2, 1e-16)


def _chunk_update(i, c, q, qn2, ptr, keys, best_v, best_b):
    base = c * _CHUNK + i * _B
    s = _score(keys, q, qn2)
    gidx = base + jax.lax.iota(jnp.int32, _B)
    m = jnp.max(jnp.where(gidx < ptr, s, -jnp.inf))
    better = (m > best_v[0]) | ((m == best_v[0]) & (base < best_b[0]))

    @pl.when(better)
    def _():
        best_v[0] = m
        best_b[0] = base


def _scan_body(ptr_ref, q_ref, *rest):
    mrefs = rest[:_NC]
    blk_ref, best_v, best_b = rest[_NC], rest[_NC + 1], rest[_NC + 2]
    i = pl.program_id(0)

    @pl.when(i == 0)
    def _():
        best_v[0] = -jnp.inf
        best_b[0] = 0

    q = q_ref[...]
    qn2 = jnp.sum(q * q)
    ptr = ptr_ref[0]
    for c, mref in enumerate(mrefs):
        _chunk_update(i, c, q, qn2, ptr, mref[...], best_v, best_b)

    @pl.when(i == pl.num_programs(0) - 1)
    def _():
        blk_ref[0] = best_b[0]


def _pick_body(s_ref, q_ref, keys_ref, mem_ref, out_ref, vsem):
    base = s_ref[0]
    ptr = s_ref[1]
    q = q_ref[...]
    qn2 = jnp.sum(q * q)
    s = _score(keys_ref[...], q, qn2)
    gidx = base + jax.lax.iota(jnp.int32, _B)
    s = jnp.where(gidx < ptr, s, -jnp.inf)
    li = jnp.argmax(s).astype(jnp.int32)
    gi = base + li
    copy = pltpu.make_async_copy(
        mem_ref.at[pl.ds(gi, 1), pl.ds(_D, _D)], out_ref, vsem)
    copy.start()
    copy.wait()


def kernel(query, memory, pointer):
    q2 = query.reshape(1, _D)
    ptr = jnp.asarray(pointer, jnp.int32).reshape(1)

    def _mspec(c):
        nblk = _CHUNK // _B
        return pl.BlockSpec((_B, _D), lambda i, p, c=c: (c * nblk + i, 0))

    blk = pl.pallas_call(
        _scan_body,
        grid_spec=pltpu.PrefetchScalarGridSpec(
            num_scalar_prefetch=1,
            grid=(_NB,),
            in_specs=[pl.BlockSpec((1, _D), lambda i, p: (0, 0))]
            + [_mspec(c) for c in range(_NC)],
            out_specs=pl.BlockSpec(memory_space=pltpu.SMEM),
            scratch_shapes=[
                pltpu.SMEM((1,), jnp.float32),
                pltpu.SMEM((1,), jnp.int32),
            ],
        ),
        out_shape=jax.ShapeDtypeStruct((1,), jnp.int32),
    )(ptr, q2, *([memory] * _NC))

    sarg = jnp.concatenate([blk, ptr])
    row = pl.pallas_call(
        _pick_body,
        grid_spec=pltpu.PrefetchScalarGridSpec(
            num_scalar_prefetch=1,
            grid=(1,),
            in_specs=[
                pl.BlockSpec((1, _D), lambda i, s: (0, 0)),
                pl.BlockSpec((_B, _D), lambda i, s: (s[0] // _B, 0)),
                pl.BlockSpec(memory_space=pl.ANY),
            ],
            out_specs=pl.BlockSpec((1, _D), lambda i, s: (0, 0)),
            scratch_shapes=[pltpu.SemaphoreType.DMA],
        ),
        out_shape=jax.ShapeDtypeStruct((1, _D), jnp.float32),
    )(sarg, q2, memory, memory)

    return row.reshape(_D)
